# baseline (speedup 1.0000x reference)
import functools

import jax
import jax.numpy as jnp
import numpy as np
from jax import lax
from jax.experimental import pallas as pl
from jax.experimental.pallas import tpu as pltpu
from jax.experimental.pallas import tpu_sc as plsc

N = 10000
E = 320000
NG = 64
D = 120
DP = 128
NB = 2000
GRID_N = N // NB
EPAD = E
EB = 5000
GRID_E = EPAD // EB
RC = 8.0
DEPTH_DC = 3

f32 = jnp.float32

_INTERPRET = False


def _pad2(w, r, c):
    return jnp.pad(w, ((0, r - w.shape[0]), (0, c - w.shape[1])))


def _pad1(b, n):
    return jnp.pad(b, (0, n - b.shape[0]))


def _onehot(gid_blk, nb):
    io = lax.broadcasted_iota(jnp.int32, (nb, NG), 1)
    return (gid_blk == io).astype(f32)


def _mm(a, b):
    return jax.lax.dot_general(a, b, (((1,), (0,)), ((), ())),
                               preferred_element_type=f32)


def _mmT(a, w):
    return jax.lax.dot_general(a, w, (((1,), (1,)), ((), ())),
                               preferred_element_type=f32)


def _segmm(oh, x):
    return jax.lax.dot_general(oh, x, (((0,), (0,)), ((), ())),
                               preferred_element_type=f32)


def _init_body(x_ref, g_ref, w_ref, b_ref, va_ref, seg_ref, cnt_ref):
    i = pl.program_id(0)
    va = _mmT(x_ref[...], w_ref[...]) + b_ref[...]
    va_ref[...] = va
    oh = _onehot(g_ref[...], NB)

    @pl.when(i == 0)
    def _():
        seg_ref[...] = jnp.zeros_like(seg_ref)
        cnt_ref[...] = jnp.zeros_like(cnt_ref)

    seg_ref[...] += _segmm(oh, va)
    c = jnp.sum(oh, axis=0)[:, None]
    cnt_ref[...] += jnp.broadcast_to(c, (NG, DP))


def _sinit_body(seg_ref, cnt_ref, w_ref, b_ref, s_ref):
    cnt = jnp.maximum(cnt_ref[...], 1.0)
    mean = seg_ref[...] / cnt
    s_ref[...] = jnp.tanh(_mmT(mean, w_ref[...]) + b_ref[...])


def _edge_body(e_ref, r_ref, wb_ref, bb_ref, wk1_ref, bk1_ref, wk2_ref,
               bk2_ref, rs_ref, sg_ref, k1_ref, k2_ref, fv_ref):
    e = e_ref[...]
    m1 = _mm(wk1_ref[...], wb_ref[...])
    m2 = _mm(wk2_ref[...], wb_ref[...])
    c1 = bk1_ref[...] + _mmT(bb_ref[...], wk1_ref[...])
    c2 = bk2_ref[...] + _mmT(bb_ref[...], wk2_ref[...])
    k1_ref[...] = _mmT(e, m1) + c1
    k2_ref[...] = _mmT(e, m2) + c2
    r = r_ref[...]
    rs = rs_ref[0, 0]
    sg = sg_ref[0, 0]
    f = jnp.exp(-jnp.square(r - rs) / (sg * sg)) * 0.5 * jnp.cos(
        np.float32(np.pi / RC) * r)
    fv_ref[...] = jnp.broadcast_to(jnp.where(r < RC, f, 0.0), (EB, 16))


def _super1_body(s_ref, wa_ref, ba_ref, wbp_ref, bbp_ref, wc_ref, bc_ref,
                 s2s_ref, dsup_ref, s2m64_ref):
    s = s_ref[...]
    s2s_ref[...] = jnp.tanh(_mmT(s, wa_ref[...]) + ba_ref[...])
    dsup_ref[...] = jnp.tanh(_mmT(s, wbp_ref[...]) + bbp_ref[...])
    s2m64_ref[...] = jnp.tanh(_mmT(s, wc_ref[...]) + bc_ref[...])


def _heads_body(v_ref, g_ref, dsup_ref, wap_ref, bap_ref, wcbd_ref, bc8_ref,
                wdp_ref, bdp_ref, lg_ref, dv_ref, den_ref):
    i = pl.program_id(0)
    v = v_ref[...]
    oh = _onehot(g_ref[...], NB)
    dn = jnp.tanh(_mmT(v, wap_ref[...]) + bap_ref[...])
    supb = _mm(oh, dsup_ref[...])
    lg = _mm(dn * supb, wcbd_ref[...]) + bc8_ref[...]
    lg_ref[...] = lg
    dv_ref[...] = _mmT(v, wdp_ref[...]) + bdp_ref[...]

    @pl.when(i == 0)
    def _():
        den_ref[...] = jnp.zeros_like(den_ref)

    den_ref[...] += _segmm(oh, jnp.exp(lg))


def _apply_body(lg_ref, dv_ref, den_ref, g_ref, heads_ref):
    i = pl.program_id(0)
    oh = _onehot(g_ref[...], NB)
    denb = _mm(oh, den_ref[...])
    a = jnp.exp(lg_ref[...]) / denb
    ax = jnp.concatenate(
        [jnp.broadcast_to(a[:, h:h + 1], (NB, DP)) for h in range(4)], axis=1)
    hh = dv_ref[...] * ax

    @pl.when(i == 0)
    def _():
        heads_ref[...] = jnp.zeros_like(heads_ref)

    heads_ref[...] += _segmm(oh, hh)


def _gru_math(gi, gh, h):
    r = jax.nn.sigmoid(gi[:, :DP] + gh[:, :DP])
    z = jax.nn.sigmoid(gi[:, DP:2 * DP] + gh[:, DP:2 * DP])
    n = jnp.tanh(gi[:, 2 * DP:] + r * gh[:, 2 * DP:])
    return (1.0 - z) * n + z * h


def _super2_body(heads_ref, s2s_ref, s_ref, wb_ref, bb_ref, wga_ref, wgb_ref,
                 bz_ref, wih_ref, bih_ref, whh_ref, bhh_ref, ss_ref):
    s2s = s2s_ref[...]
    s = s_ref[...]
    m2s = jnp.tanh(_mmT(heads_ref[...], wb_ref[...]) + bb_ref[...])
    z = jax.nn.sigmoid(_mmT(s2s, wga_ref[...]) + _mmT(m2s, wgb_ref[...])
                       + bz_ref[...])
    h = z * m2s + (1.0 - z) * s2s
    gi = _mmT(s, wih_ref[...]) + bih_ref[...]
    gh = _mmT(h, whh_ref[...]) + bhh_ref[...]
    ss_ref[...] = _gru_math(gi, gh, h)


def _update_body(v_ref, p0_ref, p1_ref, g_ref, s2m64_ref, we1_ref, we2_ref,
                 be_ref, wga_ref, wgb_ref, bz_ref, wih_ref, bih_ref, whh_ref,
                 bhh_ref, vv_ref):
    v = v_ref[...]
    sve = p0_ref[...] + p1_ref[...]
    m2m = _mmT(sve, we1_ref[...]) + _mmT(v, we2_ref[...]) + be_ref[...]
    m2m = jnp.maximum(m2m, 0.1 * m2m)
    oh = _onehot(g_ref[...], NB)
    s2m = _mm(oh, s2m64_ref[...])
    z = jax.nn.sigmoid(_mmT(m2m, wga_ref[...]) + _mmT(s2m, wgb_ref[...])
                       + bz_ref[...])
    h = z * s2m + (1.0 - z) * m2m
    gi = _mmT(v, wih_ref[...]) + bih_ref[...]
    gh = _mmT(h, whh_ref[...]) + bhh_ref[...]
    vv_ref[...] = _gru_math(gi, gh, h)


def _dcnode_body(p0_ref, p1_ref, v0_ref, wa_ref, ba_ref, vc_ref):
    x = p0_ref[...] + p1_ref[...] + v0_ref[...]
    vc_ref[...] = jnp.maximum(_mmT(x, wa_ref[...]) + ba_ref[...], 0.0)


def _pool_body(vd_ref, va_ref, g_ref, sumd_ref, suma_ref):
    i = pl.program_id(0)
    oh = _onehot(g_ref[...], NB)

    @pl.when(i == 0)
    def _():
        sumd_ref[...] = jnp.zeros_like(sumd_ref)
        suma_ref[...] = jnp.zeros_like(suma_ref)

    sumd_ref[...] += _segmm(oh, vd_ref[...])
    suma_ref[...] += _segmm(oh, va_ref[...])


def _cls_body(sumd_ref, s_ref, suma_ref, cnt_ref, vina_ref, w1a_ref, w1b_ref,
              w1c_ref, w1d_ref, b1_ref, w2_ref, b2_ref, w3_ref, b3_ref,
              p1_ref, p2_ref, out_ref):
    cnt = jnp.maximum(cnt_ref[...], 1.0)
    mean_v = suma_ref[...] / cnt
    x = (_mmT(sumd_ref[...], w1a_ref[...]) + _mmT(s_ref[...], w1b_ref[...])
         + _mmT(mean_v, w1c_ref[...]) + _mmT(vina_ref[...], w1d_ref[...])
         + b1_ref[...])
    p1 = p1_ref[0, 0]
    p2 = p2_ref[0, 0]
    x = jnp.where(x >= 0, x, p1 * x)
    x = _mmT(x, w2_ref[...]) + b2_ref[...]
    x = jnp.where(x >= 0, x, p2 * x)
    out_ref[...] = _mmT(x, w3_ref[...]) + b3_ref[...]


def _vspec(bs=None, imap=None, smem=False):
    if smem:
        return pl.BlockSpec(memory_space=pltpu.SMEM)
    return pl.BlockSpec(bs, imap)


_W0 = lambda i: (0, 0)
_ROW = lambda i: (i, 0)


def _call(body, grid, in_specs, out_specs, out_shape, args):
    return pl.pallas_call(
        body, grid=grid, in_specs=in_specs, out_specs=out_specs,
        out_shape=out_shape, interpret=_INTERPRET)(*args)


SC_NC = 2
SC_NS = 16
SC_NT = SC_NC * SC_NS
CH = 80
EPT = EPAD // SC_NT
NCHUNK = EPT // CH
NPAD = 10240
RPT = NPAD // SC_NS


def _make_sc_edge(mode):
    mesh = plsc.VectorSubcoreMesh(core_axis_name="c", subcore_axis_name="s")
    kscr = (pltpu.VMEM((CH, DP), f32) if mode == "vec"
            else pltpu.VMEM((CH, 16), f32))

    @functools.partial(
        pl.kernel,
        out_type=jax.ShapeDtypeStruct((SC_NC, NPAD, DP), f32),
        mesh=mesh,
        scratch_types=[
            pltpu.VMEM((CH, DP), f32),
            pltpu.VMEM((CH, DP), f32),
            kscr,
            kscr,
            pltpu.VMEM((CH,), jnp.int32),
            pltpu.VMEM((CH,), jnp.int32),
            pltpu.VMEM((1, CH), jnp.int32),
            pltpu.VMEM((1, CH), jnp.int32),
            pltpu.VMEM_SHARED((NPAD, DP), f32),
            pltpu.SemaphoreType.DMA,
            pltpu.SemaphoreType.DMA,
            pltpu.SemaphoreType.DMA,
            pltpu.SemaphoreType.DMA,
        ],
    )
    def sc_edge(v_hbm, k_hbm, src_hbm, dst_hbm, zero_hbm, out_hbm,
                gra, grb, kra, krb, sia, sib, dia, dib, acc,
                sga, sgb, ska, skb):
        cid = lax.axis_index("c")
        sid = lax.axis_index("s")
        wid = sid * SC_NC + cid
        pltpu.sync_copy(zero_hbm.at[pl.ds(sid * RPT, RPT)],
                        acc.at[pl.ds(sid * RPT, RPT)])
        plsc.subcore_barrier()
        base0 = wid * EPT

        def load(j, gb, kb, sb, db, sg, sk):
            base = base0 + j * CH
            pltpu.sync_copy(src_hbm.at[pl.ds(base, CH)], sb)
            pltpu.sync_copy(dst_hbm.at[pl.ds(base, CH)], db.at[0])
            pltpu.async_copy(v_hbm.at[sb], gb, sg)
            pltpu.async_copy(k_hbm.at[pl.ds(base, CH)], kb, sk)

        def consume(gb, kb, sb, db, sg, sk):
            pltpu.make_async_copy(v_hbm.at[sb], gb, sg).wait()
            pltpu.make_async_copy(k_hbm.at[pl.ds(0, CH)], kb, sk).wait()

            def row(i, c2):
                if mode == "scal":
                    fs = kb[i, pl.ds(0, 16)]
                for t in range(DP // 16):
                    sl = pl.ds(t * 16, 16)
                    if mode == "vec":
                        x = gb[i, sl] * kb[i, sl]
                        x = jnp.maximum(x, 0.1 * x)
                    else:
                        x = gb[i, sl] * fs
                    gb[i, sl] = x
                return c2

            lax.fori_loop(0, CH, row, 0, unroll=4)
            pltpu.sync_copy(gb, acc.at[db.at[0]], add=True)

        bufa = (gra, kra, sia, dia, sga, ska)
        bufb = (grb, krb, sib, dib, sgb, skb)
        load(0, *bufa)

        def pair(p, carry):
            load(2 * p + 1, *bufb)
            consume(*bufa)
            load(2 * p + 2, *bufa)
            consume(*bufb)
            return carry

        lax.fori_loop(0, (NCHUNK - 1) // 2, pair, 0, unroll=False)
        consume(*bufa)
        plsc.subcore_barrier()
        pltpu.sync_copy(acc.at[pl.ds(sid * RPT, RPT)],
                        out_hbm.at[cid, pl.ds(sid * RPT, RPT)])

    return sc_edge


_get_sc_edge = functools.lru_cache(maxsize=None)(_make_sc_edge)


def _sc_edge_pass(v, k, src, dst, zeros, mode):
    if _INTERPRET:
        g = v[src]
        if mode == "vec":
            ve = g * k
            ve = jnp.maximum(ve, 0.1 * ve)
        else:
            ve = g * k[:, :1]
        s = jax.ops.segment_sum(ve, dst, num_segments=N)
        return s, jnp.zeros_like(s)
    parts = _get_sc_edge(mode)(v, k, src, dst, zeros)
    return parts[0, :N], parts[1, :N]


def _gru_w(w):
    return jnp.concatenate(
        [_pad2(w[D * t:D * (t + 1)], DP, DP) for t in range(3)], axis=0)


def _gru_b(b):
    return jnp.concatenate(
        [_pad1(b[D * t:D * (t + 1)], DP) for t in range(3)])[None]


def _layer_weights(p, c):
    lw = {}
    lw["wA"] = _pad2(p[c + "_A_w"], DP, DP)
    lw["bA"] = _pad1(p[c + "_A_b"], DP)[None]
    lw["wBp"] = jnp.concatenate(
        [_pad2(p["%s_h%d_B_w" % (c, h)], DP, DP) for h in range(4)], axis=0)
    lw["bBp"] = jnp.concatenate(
        [_pad1(p["%s_h%d_B_b" % (c, h)], DP) for h in range(4)])[None]
    lw["wC"] = _pad2(p[c + "_C_w"], DP, DP)
    lw["bC"] = _pad1(p[c + "_C_b"], DP)[None]
    lw["wAp"] = jnp.concatenate(
        [_pad2(p["%s_h%d_A_w" % (c, h)], DP, DP) for h in range(4)], axis=0)
    lw["bAp"] = jnp.concatenate(
        [_pad1(p["%s_h%d_A_b" % (c, h)], DP) for h in range(4)])[None]
    wcbd = jnp.zeros((4 * DP, 8), f32)
    bc8 = jnp.zeros((8,), f32)
    for h in range(4):
        wcbd = wcbd.at[h * DP:h * DP + D, h].set(p["%s_h%d_C_w" % (c, h)][0])
        bc8 = bc8.at[h].set(p["%s_h%d_C_b" % (c, h)][0])
    lw["wCbd"] = wcbd
    lw["bC8"] = bc8[None]
    lw["wDp"] = jnp.concatenate(
        [_pad2(p["%s_h%d_D_w" % (c, h)], DP, DP) for h in range(4)], axis=0)
    lw["bDp"] = jnp.concatenate(
        [_pad1(p["%s_h%d_D_b" % (c, h)], DP) for h in range(4)])[None]
    wb = jnp.zeros((DP, 4 * DP), f32)
    for h in range(4):
        wb = wb.at[:D, h * DP:h * DP + D].set(
            p[c + "_B_w"][:, h * D:(h + 1) * D])
    lw["wBm"] = wb
    lw["bBm"] = _pad1(p[c + "_B_b"], DP)[None]
    ew = p[c + "_E_w"]
    lw["wE1"] = _pad2(ew[:, :D], DP, DP)
    lw["wE2"] = _pad2(ew[:, D:], DP, DP)
    lw["bE"] = _pad1(p[c + "_E_b"], DP)[None]
    for g in ("gm", "gs"):
        pre = c + "_" + g
        lw[g + "_wA"] = _pad2(p[pre + "_A_w"], DP, DP)
        lw[g + "_wB"] = _pad2(p[pre + "_B_w"], DP, DP)
        lw[g + "_bz"] = _pad1(p[pre + "_A_b"] + p[pre + "_B_b"], DP)[None]
        lw[g + "_wih"] = _gru_w(p[pre + "_Wih"])
        lw[g + "_bih"] = _gru_b(p[pre + "_bih"])
        lw[g + "_whh"] = _gru_w(p[pre + "_Whh"])
        lw[g + "_bhh"] = _gru_b(p[pre + "_bhh"])
    return lw


def kernel(x_a, e_a, r_dist, vina, params, edge_index, graph_ids):
    p = params
    src = edge_index[0]
    dst = edge_index[1]
    gid2 = graph_ids.reshape(N, 1)
    x_pad = jnp.pad(x_a, ((0, 0), (0, DP - x_a.shape[1])))
    e_pad = jnp.pad(e_a, ((0, 0), (0, 16 - e_a.shape[1])))
    r_pad = r_dist
    vina_pad = _pad2(vina, NG, DP)
    zeros = jnp.zeros((NPAD, DP), f32)

    wa = _pad2(p["a_init_w"], DP, DP)
    ba = _pad1(p["a_init_b"], DP)[None]
    wm = _pad2(p["mA_w"], DP, DP)
    bm = _pad1(p["mA_b"], DP)[None]
    wb8 = _pad2(p["b_init_w"], 8, 16)
    bb8 = _pad1(p["b_init_b"], 8)[None]
    wk1 = _pad2(p["c1_K_w"], DP, 8)
    bk1 = _pad1(p["c1_K_b"], DP)[None]
    wk2 = _pad2(p["c2_K_w"], DP, 8)
    bk2 = _pad1(p["c2_K_b"], DP)[None]

    nrow = pl.BlockSpec((NB, DP), _ROW)
    nrow4 = pl.BlockSpec((NB, 4 * DP), _ROW)
    nrow8 = pl.BlockSpec((NB, 8), _ROW)
    grow = pl.BlockSpec((NB, 1), _ROW)
    w128 = pl.BlockSpec((DP, DP), _W0)
    b128 = pl.BlockSpec((1, DP), _W0)
    w512r = pl.BlockSpec((4 * DP, DP), _W0)
    b512 = pl.BlockSpec((1, 4 * DP), _W0)
    g64 = pl.BlockSpec((NG, DP), _W0)
    g64x4 = pl.BlockSpec((NG, 4 * DP), _W0)
    g64x8 = pl.BlockSpec((NG, 8), _W0)
    smem = pl.BlockSpec(memory_space=pltpu.SMEM)

    va, seg0, cnt = _call(
        _init_body, (GRID_N,),
        [nrow, grow, w128, b128],
        [nrow, g64, g64],
        [jax.ShapeDtypeStruct((N, DP), f32),
         jax.ShapeDtypeStruct((NG, DP), f32),
         jax.ShapeDtypeStruct((NG, DP), f32)],
        (x_pad, gid2, wa, ba))

    s = _call(
        _sinit_body, (1,),
        [g64, g64, w128, b128],
        g64,
        jax.ShapeDtypeStruct((NG, DP), f32),
        (seg0, cnt, wm, bm))

    erow = pl.BlockSpec((EB, 16), _ROW)
    rrow = pl.BlockSpec((EB, 1), _ROW)
    krow = pl.BlockSpec((EB, DP), _ROW)
    k1, k2, fv = _call(
        _edge_body, (GRID_E,),
        [erow, rrow, pl.BlockSpec((8, 16), _W0), pl.BlockSpec((1, 8), _W0),
         pl.BlockSpec((DP, 8), _W0), b128, pl.BlockSpec((DP, 8), _W0), b128,
         smem, smem],
        [krow, krow, erow],
        [jax.ShapeDtypeStruct((EPAD, DP), f32),
         jax.ShapeDtypeStruct((EPAD, DP), f32),
         jax.ShapeDtypeStruct((EPAD, 16), f32)],
        (e_pad, r_pad, wb8, bb8, wk1, bk1, wk2, bk2,
         p["dc_rs"].reshape(1, 1), p["dc_sigma"].reshape(1, 1)))

    for c, kc in (("c1", k1), ("c2", k2)):
        lw = _layer_weights(p, c)
        s2s, dsup, s2m64 = _call(
            _super1_body, (1,),
            [g64, w128, b128, w512r, b512, w128, b128],
            [g64, g64x4, g64],
            [jax.ShapeDtypeStruct((NG, DP), f32),
             jax.ShapeDtypeStruct((NG, 4 * DP), f32),
             jax.ShapeDtypeStruct((NG, DP), f32)],
            (s, lw["wA"], lw["bA"], lw["wBp"], lw["bBp"], lw["wC"], lw["bC"]))

        lg, dv, den = _call(
            _heads_body, (GRID_N,),
            [nrow, grow, g64x4, w512r, b512,
             pl.BlockSpec((4 * DP, 8), _W0), pl.BlockSpec((1, 8), _W0),
             w512r, b512],
            [nrow8, nrow4, g64x8],
            [jax.ShapeDtypeStruct((N, 8), f32),
             jax.ShapeDtypeStruct((N, 4 * DP), f32),
             jax.ShapeDtypeStruct((NG, 8), f32)],
            (va, gid2, dsup, lw["wAp"], lw["bAp"], lw["wCbd"], lw["bC8"],
             lw["wDp"], lw["bDp"]))

        heads = _call(
            _apply_body, (GRID_N,),
            [nrow8, nrow4, g64x8, grow],
            g64x4,
            jax.ShapeDtypeStruct((NG, 4 * DP), f32),
            (lg, dv, den, gid2))

        p0, p1 = _sc_edge_pass(va, kc, src, dst, zeros, mode="vec")

        va = _call(
            _update_body, (GRID_N,),
            [nrow, nrow, nrow, grow, g64, w128, w128, b128, w128, w128, b128,
             pl.BlockSpec((3 * DP, DP), _W0), pl.BlockSpec((1, 3 * DP), _W0),
             pl.BlockSpec((3 * DP, DP), _W0), pl.BlockSpec((1, 3 * DP), _W0)],
            nrow,
            jax.ShapeDtypeStruct((N, DP), f32),
            (va, p0, p1, gid2, s2m64, lw["wE1"], lw["wE2"], lw["bE"],
             lw["gm_wA"], lw["gm_wB"], lw["gm_bz"],
             lw["gm_wih"], lw["gm_bih"], lw["gm_whh"], lw["gm_bhh"]))

        s = _call(
            _super2_body, (1,),
            [g64x4, g64, g64, pl.BlockSpec((DP, 4 * DP), _W0), b128,
             w128, w128, b128,
             pl.BlockSpec((3 * DP, DP), _W0), pl.BlockSpec((1, 3 * DP), _W0),
             pl.BlockSpec((3 * DP, DP), _W0), pl.BlockSpec((1, 3 * DP), _W0)],
            g64,
            jax.ShapeDtypeStruct((NG, DP), f32),
            (heads, s2s, s, lw["wBm"], lw["bBm"],
             lw["gs_wA"], lw["gs_wB"], lw["gs_bz"],
             lw["gs_wih"], lw["gs_bih"], lw["gs_whh"], lw["gs_bhh"]))

    wdc = _pad2(p["dc_A_w"], DP, DP)
    bdc = _pad1(p["dc_A_b"], DP)[None]
    v0 = va
    vc = va
    for _ in range(DEPTH_DC):
        q0, q1 = _sc_edge_pass(vc, fv, src, dst, zeros, mode="scal")
        vc = _call(
            _dcnode_body, (GRID_N,),
            [nrow, nrow, nrow, w128, b128],
            nrow,
            jax.ShapeDtypeStruct((N, DP), f32),
            (q0, q1, v0, wdc, bdc))

    sumd, suma = _call(
        _pool_body, (GRID_N,),
        [nrow, nrow, grow],
        [g64, g64],
        [jax.ShapeDtypeStruct((NG, DP), f32),
         jax.ShapeDtypeStruct((NG, DP), f32)],
        (vc, va, gid2))

    w1 = p["cls1_w"]
    w1a = _pad2(w1[:, :D], 384, DP)
    w1b = _pad2(w1[:, D:2 * D], 384, DP)
    w1c = _pad2(w1[:, 2 * D:3 * D], 384, DP)
    w1d = _pad2(w1[:, 3 * D:], 384, DP)
    b1 = _pad1(p["cls1_b"], 384)[None]
    w2 = _pad2(p["cls2_w"], 256, 384)
    b2 = _pad1(p["cls2_b"], 256)[None]
    w3 = _pad2(p["cls3_w"], 8, 256)
    b3 = _pad1(p["cls3_b"], 8)[None]

    out8 = _call(
        _cls_body, (1,),
        [g64, g64, g64, g64, g64,
         pl.BlockSpec((384, DP), _W0), pl.BlockSpec((384, DP), _W0),
         pl.BlockSpec((384, DP), _W0), pl.BlockSpec((384, DP), _W0),
         pl.BlockSpec((1, 384), _W0),
         pl.BlockSpec((256, 384), _W0), pl.BlockSpec((1, 256), _W0),
         pl.BlockSpec((8, 256), _W0), pl.BlockSpec((1, 8), _W0),
         smem, smem],
        pl.BlockSpec((NG, 8), _W0),
        jax.ShapeDtypeStruct((NG, 8), f32),
        (sumd, s, suma, cnt, vina_pad, w1a, w1b, w1c, w1d, b1, w2, b2, w3, b3,
         p["cls_p1"].reshape(1, 1), p["cls_p2"].reshape(1, 1)))

    return out8[:, :1]


# speedup vs baseline: 1.3925x; 1.3925x over previous
import functools

import jax
import jax.numpy as jnp
import numpy as np
from jax import lax
from jax.experimental import pallas as pl
from jax.experimental.pallas import tpu as pltpu
from jax.experimental.pallas import tpu_sc as plsc

N = 10000
E = 320000
NG = 64
D = 120
DP = 128
NB = 2000
GRID_N = N // NB
EPAD = E
EB = 5000
GRID_E = EPAD // EB
RC = 8.0
DEPTH_DC = 3

f32 = jnp.float32

_INTERPRET = False


def _pad2(w, r, c):
    return jnp.pad(w, ((0, r - w.shape[0]), (0, c - w.shape[1])))


def _pad1(b, n):
    return jnp.pad(b, (0, n - b.shape[0]))


def _onehot(gid_blk, nb):
    io = lax.broadcasted_iota(jnp.int32, (nb, NG), 1)
    return (gid_blk == io).astype(f32)


def _mm(a, b):
    return jax.lax.dot_general(a, b, (((1,), (0,)), ((), ())),
                               preferred_element_type=f32)


def _mmT(a, w):
    return jax.lax.dot_general(a, w, (((1,), (1,)), ((), ())),
                               preferred_element_type=f32)


def _segmm(oh, x):
    return jax.lax.dot_general(oh, x, (((0,), (0,)), ((), ())),
                               preferred_element_type=f32)


def _init_body(x_ref, g_ref, w_ref, b_ref, va_ref, seg_ref, cnt_ref):
    i = pl.program_id(0)
    va = _mmT(x_ref[...], w_ref[...]) + b_ref[...]
    va_ref[...] = va
    oh = _onehot(g_ref[...], NB)

    @pl.when(i == 0)
    def _():
        seg_ref[...] = jnp.zeros_like(seg_ref)
        cnt_ref[...] = jnp.zeros_like(cnt_ref)

    seg_ref[...] += _segmm(oh, va)
    c = jnp.sum(oh, axis=0)[:, None]
    cnt_ref[...] += jnp.broadcast_to(c, (NG, DP))


def _sinit_body(seg_ref, cnt_ref, w_ref, b_ref, s_ref):
    cnt = jnp.maximum(cnt_ref[...], 1.0)
    mean = seg_ref[...] / cnt
    s_ref[...] = jnp.tanh(_mmT(mean, w_ref[...]) + b_ref[...])


def _edge_body(e_ref, r_ref, wb_ref, bb_ref, wk1_ref, bk1_ref, wk2_ref,
               bk2_ref, rs_ref, sg_ref, k1_ref, k2_ref, fv_ref):
    e = e_ref[...]
    m1 = _mm(wk1_ref[...], wb_ref[...])
    m2 = _mm(wk2_ref[...], wb_ref[...])
    c1 = bk1_ref[...] + _mmT(bb_ref[...], wk1_ref[...])
    c2 = bk2_ref[...] + _mmT(bb_ref[...], wk2_ref[...])
    k1_ref[...] = _mmT(e, m1) + c1
    k2_ref[...] = _mmT(e, m2) + c2
    r = r_ref[...]
    rs = rs_ref[0, 0]
    sg = sg_ref[0, 0]
    f = jnp.exp(-jnp.square(r - rs) / (sg * sg)) * 0.5 * jnp.cos(
        np.float32(np.pi / RC) * r)
    fv_ref[...] = jnp.broadcast_to(jnp.where(r < RC, f, 0.0), (EB, 16))


def _super1_body(s_ref, wa_ref, ba_ref, wbp_ref, bbp_ref, wc_ref, bc_ref,
                 s2s_ref, dsup_ref, s2m64_ref):
    s = s_ref[...]
    s2s_ref[...] = jnp.tanh(_mmT(s, wa_ref[...]) + ba_ref[...])
    dsup_ref[...] = jnp.tanh(_mmT(s, wbp_ref[...]) + bbp_ref[...])
    s2m64_ref[...] = jnp.tanh(_mmT(s, wc_ref[...]) + bc_ref[...])


def _heads_body(v_ref, g_ref, dsup_ref, wap_ref, bap_ref, wcbd_ref, bc8_ref,
                wdp_ref, bdp_ref, lg_ref, dv_ref, den_ref):
    i = pl.program_id(0)
    v = v_ref[...]
    oh = _onehot(g_ref[...], NB)
    dn = jnp.tanh(_mmT(v, wap_ref[...]) + bap_ref[...])
    supb = _mm(oh, dsup_ref[...])
    lg = _mm(dn * supb, wcbd_ref[...]) + bc8_ref[...]
    lg_ref[...] = lg
    dv_ref[...] = _mmT(v, wdp_ref[...]) + bdp_ref[...]

    @pl.when(i == 0)
    def _():
        den_ref[...] = jnp.zeros_like(den_ref)

    den_ref[...] += _segmm(oh, jnp.exp(lg))


def _apply_body(lg_ref, dv_ref, den_ref, g_ref, heads_ref):
    i = pl.program_id(0)
    oh = _onehot(g_ref[...], NB)
    denb = _mm(oh, den_ref[...])
    a = jnp.exp(lg_ref[...]) / denb
    ax = jnp.concatenate(
        [jnp.broadcast_to(a[:, h:h + 1], (NB, DP)) for h in range(4)], axis=1)
    hh = dv_ref[...] * ax

    @pl.when(i == 0)
    def _():
        heads_ref[...] = jnp.zeros_like(heads_ref)

    heads_ref[...] += _segmm(oh, hh)


def _gru_math(gi, gh, h):
    r = jax.nn.sigmoid(gi[:, :DP] + gh[:, :DP])
    z = jax.nn.sigmoid(gi[:, DP:2 * DP] + gh[:, DP:2 * DP])
    n = jnp.tanh(gi[:, 2 * DP:] + r * gh[:, 2 * DP:])
    return (1.0 - z) * n + z * h


def _super2_body(heads_ref, s2s_ref, s_ref, wb_ref, bb_ref, wga_ref, wgb_ref,
                 bz_ref, wih_ref, bih_ref, whh_ref, bhh_ref, ss_ref):
    s2s = s2s_ref[...]
    s = s_ref[...]
    m2s = jnp.tanh(_mmT(heads_ref[...], wb_ref[...]) + bb_ref[...])
    z = jax.nn.sigmoid(_mmT(s2s, wga_ref[...]) + _mmT(m2s, wgb_ref[...])
                       + bz_ref[...])
    h = z * m2s + (1.0 - z) * s2s
    gi = _mmT(s, wih_ref[...]) + bih_ref[...]
    gh = _mmT(h, whh_ref[...]) + bhh_ref[...]
    ss_ref[...] = _gru_math(gi, gh, h)


def _update_body(v_ref, p0_ref, p1_ref, g_ref, s2m64_ref, we1_ref, we2_ref,
                 be_ref, wga_ref, wgb_ref, bz_ref, wih_ref, bih_ref, whh_ref,
                 bhh_ref, vv_ref):
    v = v_ref[...]
    sve = p0_ref[...] + p1_ref[...]
    m2m = _mmT(sve, we1_ref[...]) + _mmT(v, we2_ref[...]) + be_ref[...]
    m2m = jnp.maximum(m2m, 0.1 * m2m)
    oh = _onehot(g_ref[...], NB)
    s2m = _mm(oh, s2m64_ref[...])
    z = jax.nn.sigmoid(_mmT(m2m, wga_ref[...]) + _mmT(s2m, wgb_ref[...])
                       + bz_ref[...])
    h = z * s2m + (1.0 - z) * m2m
    gi = _mmT(v, wih_ref[...]) + bih_ref[...]
    gh = _mmT(h, whh_ref[...]) + bhh_ref[...]
    vv_ref[...] = _gru_math(gi, gh, h)


def _dcnode_body(p0_ref, p1_ref, v0_ref, wa_ref, ba_ref, vc_ref):
    x = p0_ref[...] + p1_ref[...] + v0_ref[...]
    vc_ref[...] = jnp.maximum(_mmT(x, wa_ref[...]) + ba_ref[...], 0.0)


def _pool_body(vd_ref, va_ref, g_ref, sumd_ref, suma_ref):
    i = pl.program_id(0)
    oh = _onehot(g_ref[...], NB)

    @pl.when(i == 0)
    def _():
        sumd_ref[...] = jnp.zeros_like(sumd_ref)
        suma_ref[...] = jnp.zeros_like(suma_ref)

    sumd_ref[...] += _segmm(oh, vd_ref[...])
    suma_ref[...] += _segmm(oh, va_ref[...])


def _cls_body(sumd_ref, s_ref, suma_ref, cnt_ref, vina_ref, w1a_ref, w1b_ref,
              w1c_ref, w1d_ref, b1_ref, w2_ref, b2_ref, w3_ref, b3_ref,
              p1_ref, p2_ref, out_ref):
    cnt = jnp.maximum(cnt_ref[...], 1.0)
    mean_v = suma_ref[...] / cnt
    x = (_mmT(sumd_ref[...], w1a_ref[...]) + _mmT(s_ref[...], w1b_ref[...])
         + _mmT(mean_v, w1c_ref[...]) + _mmT(vina_ref[...], w1d_ref[...])
         + b1_ref[...])
    p1 = p1_ref[0, 0]
    p2 = p2_ref[0, 0]
    x = jnp.where(x >= 0, x, p1 * x)
    x = _mmT(x, w2_ref[...]) + b2_ref[...]
    x = jnp.where(x >= 0, x, p2 * x)
    out_ref[...] = _mmT(x, w3_ref[...]) + b3_ref[...]


def _vspec(bs=None, imap=None, smem=False):
    if smem:
        return pl.BlockSpec(memory_space=pltpu.SMEM)
    return pl.BlockSpec(bs, imap)


_W0 = lambda i: (0, 0)
_ROW = lambda i: (i, 0)


def _call(body, grid, in_specs, out_specs, out_shape, args):
    return pl.pallas_call(
        body, grid=grid, in_specs=in_specs, out_specs=out_specs,
        out_shape=out_shape, interpret=_INTERPRET)(*args)


SC_NC = 2
SC_NS = 16
SC_NT = SC_NC * SC_NS
CH = 80
EPT = EPAD // SC_NT
NCHUNK = EPT // CH
NPAD = 10240
RPT = NPAD // SC_NS


def _make_sc_edge(mode):
    mesh = plsc.VectorSubcoreMesh(core_axis_name="c", subcore_axis_name="s")
    kscr = (pltpu.VMEM((CH, DP), f32) if mode == "vec"
            else pltpu.VMEM((CH, 16), f32))

    @functools.partial(
        pl.kernel,
        out_type=jax.ShapeDtypeStruct((SC_NC, NPAD, DP), f32),
        mesh=mesh,
        scratch_types=[
            pltpu.VMEM((CH, DP), f32),
            pltpu.VMEM((CH, DP), f32),
            kscr,
            kscr,
            pltpu.VMEM((CH,), jnp.int32),
            pltpu.VMEM((CH,), jnp.int32),
            pltpu.VMEM((1, CH), jnp.int32),
            pltpu.VMEM((1, CH), jnp.int32),
            pltpu.VMEM_SHARED((NPAD, DP), f32),
            pltpu.SemaphoreType.DMA,
            pltpu.SemaphoreType.DMA,
            pltpu.SemaphoreType.DMA,
            pltpu.SemaphoreType.DMA,
        ],
    )
    def sc_edge(v_hbm, k_hbm, src_hbm, dst_hbm, zero_hbm, out_hbm,
                gra, grb, kra, krb, sia, sib, dia, dib, acc,
                sga, sgb, ska, skb):
        cid = lax.axis_index("c")
        sid = lax.axis_index("s")
        wid = sid * SC_NC + cid
        pltpu.sync_copy(zero_hbm.at[pl.ds(sid * RPT, RPT)],
                        acc.at[pl.ds(sid * RPT, RPT)])
        plsc.subcore_barrier()
        base0 = wid * EPT

        def load(j, gb, kb, sb, db, sg, sk):
            base = base0 + j * CH
            pltpu.sync_copy(src_hbm.at[pl.ds(base, CH)], sb)
            pltpu.sync_copy(dst_hbm.at[pl.ds(base, CH)], db.at[0])
            pltpu.async_copy(v_hbm.at[sb], gb, sg)
            pltpu.async_copy(k_hbm.at[pl.ds(base, CH)], kb, sk)

        def consume(gb, kb, sb, db, sg, sk):
            pltpu.make_async_copy(v_hbm.at[sb], gb, sg).wait()
            pltpu.make_async_copy(k_hbm.at[pl.ds(0, CH)], kb, sk).wait()

            def row(i, c2):
                if mode == "scal":
                    fs = kb[i, pl.ds(0, 16)]
                for t in range(DP // 16):
                    sl = pl.ds(t * 16, 16)
                    if mode == "vec":
                        x = gb[i, sl] * kb[i, sl]
                        x = jnp.maximum(x, 0.1 * x)
                    else:
                        x = gb[i, sl] * fs
                    gb[i, sl] = x
                return c2

            lax.fori_loop(0, CH, row, 0, unroll=False)
            pltpu.sync_copy(gb, acc.at[db.at[0]], add=True)

        bufa = (gra, kra, sia, dia, sga, ska)
        bufb = (grb, krb, sib, dib, sgb, skb)
        load(0, *bufa)

        def pair(p, carry):
            load(2 * p + 1, *bufb)
            consume(*bufa)
            load(2 * p + 2, *bufa)
            consume(*bufb)
            return carry

        lax.fori_loop(0, (NCHUNK - 1) // 2, pair, 0, unroll=False)
        consume(*bufa)
        plsc.subcore_barrier()
        pltpu.sync_copy(acc.at[pl.ds(sid * RPT, RPT)],
                        out_hbm.at[cid, pl.ds(sid * RPT, RPT)])

    return sc_edge


_get_sc_edge = functools.lru_cache(maxsize=None)(_make_sc_edge)


def _sc_edge_pass(v, k, src, dst, zeros, mode):
    if _INTERPRET:
        g = v[src]
        if mode == "vec":
            ve = g * k
            ve = jnp.maximum(ve, 0.1 * ve)
        else:
            ve = g * k[:, :1]
        s = jax.ops.segment_sum(ve, dst, num_segments=N)
        return s, jnp.zeros_like(s)
    parts = _get_sc_edge(mode)(v, k, src, dst, zeros)
    return parts[0, :N], parts[1, :N]


def _gru_w(w):
    return jnp.concatenate(
        [_pad2(w[D * t:D * (t + 1)], DP, DP) for t in range(3)], axis=0)


def _gru_b(b):
    return jnp.concatenate(
        [_pad1(b[D * t:D * (t + 1)], DP) for t in range(3)])[None]


def _layer_weights(p, c):
    lw = {}
    lw["wA"] = _pad2(p[c + "_A_w"], DP, DP)
    lw["bA"] = _pad1(p[c + "_A_b"], DP)[None]
    lw["wBp"] = jnp.concatenate(
        [_pad2(p["%s_h%d_B_w" % (c, h)], DP, DP) for h in range(4)], axis=0)
    lw["bBp"] = jnp.concatenate(
        [_pad1(p["%s_h%d_B_b" % (c, h)], DP) for h in range(4)])[None]
    lw["wC"] = _pad2(p[c + "_C_w"], DP, DP)
    lw["bC"] = _pad1(p[c + "_C_b"], DP)[None]
    lw["wAp"] = jnp.concatenate(
        [_pad2(p["%s_h%d_A_w" % (c, h)], DP, DP) for h in range(4)], axis=0)
    lw["bAp"] = jnp.concatenate(
        [_pad1(p["%s_h%d_A_b" % (c, h)], DP) for h in range(4)])[None]
    wcbd = jnp.zeros((4 * DP, 8), f32)
    bc8 = jnp.zeros((8,), f32)
    for h in range(4):
        wcbd = wcbd.at[h * DP:h * DP + D, h].set(p["%s_h%d_C_w" % (c, h)][0])
        bc8 = bc8.at[h].set(p["%s_h%d_C_b" % (c, h)][0])
    lw["wCbd"] = wcbd
    lw["bC8"] = bc8[None]
    lw["wDp"] = jnp.concatenate(
        [_pad2(p["%s_h%d_D_w" % (c, h)], DP, DP) for h in range(4)], axis=0)
    lw["bDp"] = jnp.concatenate(
        [_pad1(p["%s_h%d_D_b" % (c, h)], DP) for h in range(4)])[None]
    wb = jnp.zeros((DP, 4 * DP), f32)
    for h in range(4):
        wb = wb.at[:D, h * DP:h * DP + D].set(
            p[c + "_B_w"][:, h * D:(h + 1) * D])
    lw["wBm"] = wb
    lw["bBm"] = _pad1(p[c + "_B_b"], DP)[None]
    ew = p[c + "_E_w"]
    lw["wE1"] = _pad2(ew[:, :D], DP, DP)
    lw["wE2"] = _pad2(ew[:, D:], DP, DP)
    lw["bE"] = _pad1(p[c + "_E_b"], DP)[None]
    for g in ("gm", "gs"):
        pre = c + "_" + g
        lw[g + "_wA"] = _pad2(p[pre + "_A_w"], DP, DP)
        lw[g + "_wB"] = _pad2(p[pre + "_B_w"], DP, DP)
        lw[g + "_bz"] = _pad1(p[pre + "_A_b"] + p[pre + "_B_b"], DP)[None]
        lw[g + "_wih"] = _gru_w(p[pre + "_Wih"])
        lw[g + "_bih"] = _gru_b(p[pre + "_bih"])
        lw[g + "_whh"] = _gru_w(p[pre + "_Whh"])
        lw[g + "_bhh"] = _gru_b(p[pre + "_bhh"])
    return lw


def kernel(x_a, e_a, r_dist, vina, params, edge_index, graph_ids):
    p = params
    src = edge_index[0]
    dst = edge_index[1]
    gid2 = graph_ids.reshape(N, 1)
    x_pad = jnp.pad(x_a, ((0, 0), (0, DP - x_a.shape[1])))
    e_pad = jnp.pad(e_a, ((0, 0), (0, 16 - e_a.shape[1])))
    r_pad = r_dist
    vina_pad = _pad2(vina, NG, DP)
    zeros = jnp.zeros((NPAD, DP), f32)

    wa = _pad2(p["a_init_w"], DP, DP)
    ba = _pad1(p["a_init_b"], DP)[None]
    wm = _pad2(p["mA_w"], DP, DP)
    bm = _pad1(p["mA_b"], DP)[None]
    wb8 = _pad2(p["b_init_w"], 8, 16)
    bb8 = _pad1(p["b_init_b"], 8)[None]
    wk1 = _pad2(p["c1_K_w"], DP, 8)
    bk1 = _pad1(p["c1_K_b"], DP)[None]
    wk2 = _pad2(p["c2_K_w"], DP, 8)
    bk2 = _pad1(p["c2_K_b"], DP)[None]

    nrow = pl.BlockSpec((NB, DP), _ROW)
    nrow4 = pl.BlockSpec((NB, 4 * DP), _ROW)
    nrow8 = pl.BlockSpec((NB, 8), _ROW)
    grow = pl.BlockSpec((NB, 1), _ROW)
    w128 = pl.BlockSpec((DP, DP), _W0)
    b128 = pl.BlockSpec((1, DP), _W0)
    w512r = pl.BlockSpec((4 * DP, DP), _W0)
    b512 = pl.BlockSpec((1, 4 * DP), _W0)
    g64 = pl.BlockSpec((NG, DP), _W0)
    g64x4 = pl.BlockSpec((NG, 4 * DP), _W0)
    g64x8 = pl.BlockSpec((NG, 8), _W0)
    smem = pl.BlockSpec(memory_space=pltpu.SMEM)

    va, seg0, cnt = _call(
        _init_body, (GRID_N,),
        [nrow, grow, w128, b128],
        [nrow, g64, g64],
        [jax.ShapeDtypeStruct((N, DP), f32),
         jax.ShapeDtypeStruct((NG, DP), f32),
         jax.ShapeDtypeStruct((NG, DP), f32)],
        (x_pad, gid2, wa, ba))

    s = _call(
        _sinit_body, (1,),
        [g64, g64, w128, b128],
        g64,
        jax.ShapeDtypeStruct((NG, DP), f32),
        (seg0, cnt, wm, bm))

    erow = pl.BlockSpec((EB, 16), _ROW)
    rrow = pl.BlockSpec((EB, 1), _ROW)
    krow = pl.BlockSpec((EB, DP), _ROW)
    k1, k2, fv = _call(
        _edge_body, (GRID_E,),
        [erow, rrow, pl.BlockSpec((8, 16), _W0), pl.BlockSpec((1, 8), _W0),
         pl.BlockSpec((DP, 8), _W0), b128, pl.BlockSpec((DP, 8), _W0), b128,
         smem, smem],
        [krow, krow, erow],
        [jax.ShapeDtypeStruct((EPAD, DP), f32),
         jax.ShapeDtypeStruct((EPAD, DP), f32),
         jax.ShapeDtypeStruct((EPAD, 16), f32)],
        (e_pad, r_pad, wb8, bb8, wk1, bk1, wk2, bk2,
         p["dc_rs"].reshape(1, 1), p["dc_sigma"].reshape(1, 1)))

    for c, kc in (("c1", k1), ("c2", k2)):
        lw = _layer_weights(p, c)
        s2s, dsup, s2m64 = _call(
            _super1_body, (1,),
            [g64, w128, b128, w512r, b512, w128, b128],
            [g64, g64x4, g64],
            [jax.ShapeDtypeStruct((NG, DP), f32),
             jax.ShapeDtypeStruct((NG, 4 * DP), f32),
             jax.ShapeDtypeStruct((NG, DP), f32)],
            (s, lw["wA"], lw["bA"], lw["wBp"], lw["bBp"], lw["wC"], lw["bC"]))

        lg, dv, den = _call(
            _heads_body, (GRID_N,),
            [nrow, grow, g64x4, w512r, b512,
             pl.BlockSpec((4 * DP, 8), _W0), pl.BlockSpec((1, 8), _W0),
             w512r, b512],
            [nrow8, nrow4, g64x8],
            [jax.ShapeDtypeStruct((N, 8), f32),
             jax.ShapeDtypeStruct((N, 4 * DP), f32),
             jax.ShapeDtypeStruct((NG, 8), f32)],
            (va, gid2, dsup, lw["wAp"], lw["bAp"], lw["wCbd"], lw["bC8"],
             lw["wDp"], lw["bDp"]))

        heads = _call(
            _apply_body, (GRID_N,),
            [nrow8, nrow4, g64x8, grow],
            g64x4,
            jax.ShapeDtypeStruct((NG, 4 * DP), f32),
            (lg, dv, den, gid2))

        p0, p1 = _sc_edge_pass(va, kc, src, dst, zeros, mode="vec")

        va = _call(
            _update_body, (GRID_N,),
            [nrow, nrow, nrow, grow, g64, w128, w128, b128, w128, w128, b128,
             pl.BlockSpec((3 * DP, DP), _W0), pl.BlockSpec((1, 3 * DP), _W0),
             pl.BlockSpec((3 * DP, DP), _W0), pl.BlockSpec((1, 3 * DP), _W0)],
            nrow,
            jax.ShapeDtypeStruct((N, DP), f32),
            (va, p0, p1, gid2, s2m64, lw["wE1"], lw["wE2"], lw["bE"],
             lw["gm_wA"], lw["gm_wB"], lw["gm_bz"],
             lw["gm_wih"], lw["gm_bih"], lw["gm_whh"], lw["gm_bhh"]))

        s = _call(
            _super2_body, (1,),
            [g64x4, g64, g64, pl.BlockSpec((DP, 4 * DP), _W0), b128,
             w128, w128, b128,
             pl.BlockSpec((3 * DP, DP), _W0), pl.BlockSpec((1, 3 * DP), _W0),
             pl.BlockSpec((3 * DP, DP), _W0), pl.BlockSpec((1, 3 * DP), _W0)],
            g64,
            jax.ShapeDtypeStruct((NG, DP), f32),
            (heads, s2s, s, lw["wBm"], lw["bBm"],
             lw["gs_wA"], lw["gs_wB"], lw["gs_bz"],
             lw["gs_wih"], lw["gs_bih"], lw["gs_whh"], lw["gs_bhh"]))

    wdc = _pad2(p["dc_A_w"], DP, DP)
    bdc = _pad1(p["dc_A_b"], DP)[None]
    v0 = va
    vc = va
    for _ in range(DEPTH_DC):
        q0, q1 = _sc_edge_pass(vc, fv, src, dst, zeros, mode="scal")
        vc = _call(
            _dcnode_body, (GRID_N,),
            [nrow, nrow, nrow, w128, b128],
            nrow,
            jax.ShapeDtypeStruct((N, DP), f32),
            (q0, q1, v0, wdc, bdc))

    sumd, suma = _call(
        _pool_body, (GRID_N,),
        [nrow, nrow, grow],
        [g64, g64],
        [jax.ShapeDtypeStruct((NG, DP), f32),
         jax.ShapeDtypeStruct((NG, DP), f32)],
        (vc, va, gid2))

    w1 = p["cls1_w"]
    w1a = _pad2(w1[:, :D], 384, DP)
    w1b = _pad2(w1[:, D:2 * D], 384, DP)
    w1c = _pad2(w1[:, 2 * D:3 * D], 384, DP)
    w1d = _pad2(w1[:, 3 * D:], 384, DP)
    b1 = _pad1(p["cls1_b"], 384)[None]
    w2 = _pad2(p["cls2_w"], 256, 384)
    b2 = _pad1(p["cls2_b"], 256)[None]
    w3 = _pad2(p["cls3_w"], 8, 256)
    b3 = _pad1(p["cls3_b"], 8)[None]

    out8 = _call(
        _cls_body, (1,),
        [g64, g64, g64, g64, g64,
         pl.BlockSpec((384, DP), _W0), pl.BlockSpec((384, DP), _W0),
         pl.BlockSpec((384, DP), _W0), pl.BlockSpec((384, DP), _W0),
         pl.BlockSpec((1, 384), _W0),
         pl.BlockSpec((256, 384), _W0), pl.BlockSpec((1, 256), _W0),
         pl.BlockSpec((8, 256), _W0), pl.BlockSpec((1, 8), _W0),
         smem, smem],
        pl.BlockSpec((NG, 8), _W0),
        jax.ShapeDtypeStruct((NG, 8), f32),
        (sumd, s, suma, cnt, vina_pad, w1a, w1b, w1c, w1d, b1, w2, b2, w3, b3,
         p["cls_p1"].reshape(1, 1), p["cls_p2"].reshape(1, 1)))

    return out8[:, :1]
